# baseline (speedup 1.0000x reference)
import jax
import jax.numpy as jnp
from jax.experimental import pallas as pl
from jax.experimental.pallas import tpu as pltpu

_NPAD = 1024
_TB = 32
_SL = 8


def _step_kernel(x_ref, U_ref, ST_ref, a_ref, b_ref, c_ref, d_ref, m_ref,
                 states_ref, vout_ref, uout_ref, spikes_ref,
                 v_scr, u_scr, irec_scr):
    g = pl.program_id(0)

    @pl.when(g == 0)
    def _init():
        v_scr[...] = jnp.zeros_like(v_scr)
        u_scr[...] = jnp.zeros_like(u_scr)

    v0 = v_scr[...]
    u0 = u_scr[...]
    U_ = U_ref[...]
    a_ = a_ref[...]
    b_ = b_ref[...]
    c_ = c_ref[...]
    d_ = d_ref[...]
    m_ = m_ref[...]

    v, u = v0, u0
    spike = (v >= 30.0).astype(jnp.float32)
    acc = spike * m_
    for i in range(_TB):
        v = jnp.where(spike > 0.0, c_, v)
        u = u + spike * d_
        I = x_ref[0, i * _SL:(i + 1) * _SL, :] * U_
        v = v + 0.5 * (0.04 * v * v + 5.0 * v + 140.0 - u + I)
        u = u + a_ * (b_ * v - u)
        state = (v >= 30.0).astype(jnp.float32)
        spikes_ref[0, i * _SL:(i + 1) * _SL, :] = spike
        states_ref[0, i * _SL:(i + 1) * _SL, :] = state
        spike = state
        if i < _TB - 1:
            acc = jnp.maximum(acc, state * m_)

    v_scr[...] = v
    u_scr[...] = u

    @pl.when(jnp.max(acc) > 0.0)
    def _redo():
        v, u = v0, u0
        spike = (v >= 30.0).astype(jnp.float32)
        ns = jnp.sum(spike * m_)
        for i in range(_TB):
            v = jnp.where(spike > 0.0, c_, v)
            u = u + spike * d_
            irec_scr[...] = jnp.zeros_like(irec_scr)

            @pl.when(ns > 0.0)
            def _matvec(spike=spike):
                row = spike.reshape(1, _SL * 128)
                irec_scr[...] = jnp.dot(
                    row, ST_ref[...],
                    preferred_element_type=jnp.float32).reshape(_SL, 128)

            I = (x_ref[0, i * _SL:(i + 1) * _SL, :] * U_
                 + irec_scr[...])
            v = v + 0.5 * (0.04 * v * v + 5.0 * v + 140.0 - u + I)
            u = u + a_ * (b_ * v - u)
            state = (v >= 30.0).astype(jnp.float32)
            spikes_ref[0, i * _SL:(i + 1) * _SL, :] = spike
            states_ref[0, i * _SL:(i + 1) * _SL, :] = state
            ns = jnp.sum(state * m_)
            spike = state
        v_scr[...] = v
        u_scr[...] = u

    vout_ref[...] = v_scr[...]
    uout_ref[...] = u_scr[...]


def kernel(data, U, S, a, b, c, d):
    T, N = data.shape
    P = _NPAD
    pad = P - N
    nblk = T // _TB

    data_p = jnp.pad(data, ((0, 0), (0, pad))).reshape(nblk, _TB * _SL, 128)
    sq = lambda x: jnp.pad(x, (0, pad)).reshape(_SL, 128)
    U_p, a_p, b_p, c_p, d_p = sq(U), sq(a), sq(b), sq(c), sq(d)
    mask_p = sq(jnp.ones((N,), jnp.float32))
    ST_p = jnp.pad(S.T, ((0, pad), (0, pad)))

    sqspec = pl.BlockSpec((_SL, 128), lambda g: (0, 0))
    per_t = pl.BlockSpec((1, _TB * _SL, 128), lambda g: (g, 0, 0))

    states, v, u, spikes = pl.pallas_call(
        _step_kernel,
        grid=(nblk,),
        in_specs=[per_t, sqspec, pl.BlockSpec((P, P), lambda g: (0, 0)),
                  sqspec, sqspec, sqspec, sqspec, sqspec],
        out_specs=[per_t, sqspec, sqspec, per_t],
        out_shape=[
            jax.ShapeDtypeStruct((nblk, _TB * _SL, 128), jnp.float32),
            jax.ShapeDtypeStruct((_SL, 128), jnp.float32),
            jax.ShapeDtypeStruct((_SL, 128), jnp.float32),
            jax.ShapeDtypeStruct((nblk, _TB * _SL, 128), jnp.float32),
        ],
        scratch_shapes=[pltpu.VMEM((_SL, 128), jnp.float32),
                        pltpu.VMEM((_SL, 128), jnp.float32),
                        pltpu.VMEM((_SL, 128), jnp.float32)],
        compiler_params=pltpu.CompilerParams(
            dimension_semantics=("arbitrary",)),
    )(data_p, U_p, ST_p, a_p, b_p, c_p, d_p, mask_p)

    return (states.reshape(T, P)[:, :N], v.reshape(P)[:N],
            u.reshape(P)[:N], spikes.reshape(T, P)[:, :N])


# speedup vs baseline: 1.0365x; 1.0365x over previous
import jax
import jax.numpy as jnp
from jax.experimental import pallas as pl
from jax.experimental.pallas import tpu as pltpu

_NPAD = 1024
_TB = 32
_SL = 8


def _step_kernel(x_ref, U_ref, ST_ref, a_ref, b_ref, c_ref, d_ref, m_ref,
                 states_ref, vout_ref, uout_ref, spikes_ref,
                 v_scr, u_scr):
    g = pl.program_id(0)

    @pl.when(g == 0)
    def _init():
        v_scr[...] = jnp.zeros_like(v_scr)
        u_scr[...] = jnp.zeros_like(u_scr)

    v0 = v_scr[...]
    u0 = u_scr[...]
    U_ = U_ref[...]
    a_ = a_ref[...]
    b_ = b_ref[...]
    c_ = c_ref[...]
    d_ = d_ref[...]
    m_ = m_ref[...]

    v, u = v0, u0
    spike = (v >= 30.0).astype(jnp.float32)
    acc = spike * m_
    for i in range(_TB):
        v = jnp.where(spike > 0.0, c_, v)
        u = u + spike * d_
        I = x_ref[0, i * _SL:(i + 1) * _SL, :] * U_
        v = v + 0.5 * (0.04 * v * v + 5.0 * v + 140.0 - u + I)
        u = u + a_ * (b_ * v - u)
        state = (v >= 30.0).astype(jnp.float32)
        spikes_ref[0, i * _SL:(i + 1) * _SL, :] = spike
        states_ref[0, i * _SL:(i + 1) * _SL, :] = state
        spike = state
        if i < _TB - 1:
            acc = jnp.maximum(acc, state * m_)

    v_scr[...] = v
    u_scr[...] = u

    @pl.when(jnp.max(acc) > 0.0)
    def _redo():
        v, u = v0, u0
        spike = (v >= 30.0).astype(jnp.float32)
        for i in range(_TB):
            v = jnp.where(spike > 0.0, c_, v)
            u = u + spike * d_
            row = spike.reshape(1, _SL * 128)
            irec = jnp.dot(row, ST_ref[...],
                           preferred_element_type=jnp.float32)
            I = (x_ref[0, i * _SL:(i + 1) * _SL, :] * U_
                 + irec.reshape(_SL, 128))
            v = v + 0.5 * (0.04 * v * v + 5.0 * v + 140.0 - u + I)
            u = u + a_ * (b_ * v - u)
            state = (v >= 30.0).astype(jnp.float32)
            spikes_ref[0, i * _SL:(i + 1) * _SL, :] = spike
            states_ref[0, i * _SL:(i + 1) * _SL, :] = state
            spike = state
        v_scr[...] = v
        u_scr[...] = u

    vout_ref[...] = v_scr[...]
    uout_ref[...] = u_scr[...]


def kernel(data, U, S, a, b, c, d):
    T, N = data.shape
    P = _NPAD
    pad = P - N
    nblk = T // _TB

    data_p = jnp.pad(data, ((0, 0), (0, pad))).reshape(nblk, _TB * _SL, 128)
    sq = lambda x: jnp.pad(x, (0, pad)).reshape(_SL, 128)
    U_p, a_p, b_p, c_p, d_p = sq(U), sq(a), sq(b), sq(c), sq(d)
    mask_p = sq(jnp.ones((N,), jnp.float32))
    ST_p = jnp.pad(S.T, ((0, pad), (0, pad)))

    sqspec = pl.BlockSpec((_SL, 128), lambda g: (0, 0))
    per_t = pl.BlockSpec((1, _TB * _SL, 128), lambda g: (g, 0, 0))

    states, v, u, spikes = pl.pallas_call(
        _step_kernel,
        grid=(nblk,),
        in_specs=[per_t, sqspec, pl.BlockSpec((P, P), lambda g: (0, 0)),
                  sqspec, sqspec, sqspec, sqspec, sqspec],
        out_specs=[per_t, sqspec, sqspec, per_t],
        out_shape=[
            jax.ShapeDtypeStruct((nblk, _TB * _SL, 128), jnp.float32),
            jax.ShapeDtypeStruct((_SL, 128), jnp.float32),
            jax.ShapeDtypeStruct((_SL, 128), jnp.float32),
            jax.ShapeDtypeStruct((nblk, _TB * _SL, 128), jnp.float32),
        ],
        scratch_shapes=[pltpu.VMEM((_SL, 128), jnp.float32),
                        pltpu.VMEM((_SL, 128), jnp.float32)],
        compiler_params=pltpu.CompilerParams(
            dimension_semantics=("arbitrary",)),
    )(data_p, U_p, ST_p, a_p, b_p, c_p, d_p, mask_p)

    return (states.reshape(T, P)[:, :N], v.reshape(P)[:N],
            u.reshape(P)[:N], spikes.reshape(T, P)[:, :N])
